# Initial kernel scaffold; baseline (speedup 1.0000x reference)
import jax
import jax.numpy as jnp
from jax.experimental import pallas as pl


def kernel(chunk_hiddens, chunk_idx, local_pos, segment_ids, n_vars):
    raise NotImplementedError("write your pallas kernel here")


# speedup vs baseline: 2.5033x; 2.5033x over previous
import functools

import jax
import jax.numpy as jnp
from jax import lax
from jax.experimental import pallas as pl
from jax.experimental.pallas import tpu as pltpu
from jax.experimental.pallas import tpu_sc as plsc

CHUNK_LEN = 8192
HIDDEN = 128
N_VARS_STATIC = 10000

NC = 2
NS = 16
NW = NC * NS

SB = 128
N_SB = 80
PER_W = N_SB * SB
NP_PAD = NW * PER_W
NV_PAD = 10240
TILE_CHUNKS = NV_PAD // (NS * SB)


def _sc_partials(table, cidx, lpos, seg):
    mesh = plsc.VectorSubcoreMesh(core_axis_name="c", subcore_axis_name="s")

    @functools.partial(
        pl.kernel,
        mesh=mesh,
        out_type=(
            jax.ShapeDtypeStruct((NC, NV_PAD, HIDDEN), jnp.float32),
            jax.ShapeDtypeStruct((NC, NV_PAD, HIDDEN), jnp.float32),
        ),
        scratch_types=[
            pltpu.VMEM((SB,), jnp.int32),
            pltpu.VMEM((SB,), jnp.int32),
            pltpu.VMEM((SB,), jnp.int32),
            pltpu.VMEM((SB, HIDDEN), jnp.float32),
            pltpu.VMEM_SHARED((NV_PAD, HIDDEN), jnp.float32),
        ],
    )
    def k(table_h, cidx_h, lpos_h, seg_h, out_s_h, out_c_h,
          fidx_v, lpos_v, seg_v, rows_v, acc_s):
        c = lax.axis_index("c")
        s = lax.axis_index("s")
        wid = c * NS + s
        base = wid * N_SB
        iota16 = lax.iota(jnp.int32, 16)

        def fill_rows(val):
            vec = jnp.full((16,), val, jnp.float32)

            def fill(r, carry):
                for v in range(HIDDEN // 16):
                    rows_v[r, pl.ds(v * 16, 16)] = vec
                return carry

            lax.fori_loop(0, SB, fill, 0)

        def make_idx(t):
            base_r = s * (TILE_CHUNKS * SB) + t * SB
            for v in range(SB // 16):
                fidx_v[pl.ds(v * 16, 16)] = iota16 + (base_r + v * 16)

        def zero_acc():
            fill_rows(0.0)
            for t in range(TILE_CHUNKS):
                make_idx(t)
                pltpu.sync_copy(rows_v, acc_s.at[fidx_v])

        def writeout(dst_h):
            for t in range(TILE_CHUNKS):
                make_idx(t)
                osl = pl.ds(s * (TILE_CHUNKS * SB) + t * SB, SB)
                pltpu.sync_copy(acc_s.at[fidx_v], rows_v)
                pltpu.sync_copy(rows_v, dst_h.at[c, osl])

        zero_acc()
        plsc.subcore_barrier()

        @pl.loop(0, N_SB)
        def body(sb):
            row = base + sb
            pltpu.sync_copy(cidx_h.at[row], fidx_v)
            pltpu.sync_copy(lpos_h.at[row], lpos_v)
            pltpu.sync_copy(seg_h.at[row], seg_v)
            for v in range(SB // 16):
                vsl = pl.ds(v * 16, 16)
                fidx_v[vsl] = fidx_v[vsl] * CHUNK_LEN + lpos_v[vsl]
            pltpu.sync_copy(table_h.at[fidx_v], rows_v)
            pltpu.sync_copy(rows_v, acc_s.at[seg_v], add=True)

        plsc.subcore_barrier()
        writeout(out_s_h)
        plsc.subcore_barrier()

        zero_acc()
        plsc.subcore_barrier()
        fill_rows(1.0)

        @pl.loop(0, N_SB)
        def body2(sb):
            pltpu.sync_copy(seg_h.at[base + sb], seg_v)
            pltpu.sync_copy(rows_v, acc_s.at[seg_v], add=True)

        plsc.subcore_barrier()
        writeout(out_c_h)

    return k(table, cidx, lpos, seg)


def _combine(sums, cnts):
    g = 16
    r = NV_PAD // g

    def body(s_ref, c_ref, o_ref):
        ssum = s_ref[0] + s_ref[1]
        csum = c_ref[0] + c_ref[1]
        o_ref[...] = ssum / jnp.maximum(csum, 1.0)

    return pl.pallas_call(
        body,
        grid=(g,),
        in_specs=[
            pl.BlockSpec((NC, r, HIDDEN), lambda i: (0, i, 0)),
            pl.BlockSpec((NC, r, HIDDEN), lambda i: (0, i, 0)),
        ],
        out_specs=pl.BlockSpec((r, HIDDEN), lambda i: (i, 0)),
        out_shape=jax.ShapeDtypeStruct((NV_PAD, HIDDEN), jnp.float32),
    )(sums, cnts)


def kernel(chunk_hiddens, chunk_idx, local_pos, segment_ids, n_vars):
    n_chunks, chunk_len, hidden = chunk_hiddens.shape
    table = chunk_hiddens.reshape(n_chunks * chunk_len, hidden)

    n_pos = chunk_idx.shape[0]
    pad = NP_PAD - n_pos
    cidx = jnp.pad(chunk_idx.astype(jnp.int32), (0, pad)).reshape(-1, SB)
    lpos = jnp.pad(local_pos.astype(jnp.int32), (0, pad)).reshape(-1, SB)
    seg = jnp.pad(
        segment_ids.astype(jnp.int32), (0, pad),
        constant_values=N_VARS_STATIC,
    ).reshape(-1, SB)

    sums, cnts = _sc_partials(table, cidx, lpos, seg)
    out = _combine(sums, cnts)
    return out[:N_VARS_STATIC]


# speedup vs baseline: 2.9901x; 1.1945x over previous
import functools

import jax
import jax.numpy as jnp
from jax import lax
from jax.experimental import pallas as pl
from jax.experimental.pallas import tpu as pltpu
from jax.experimental.pallas import tpu_sc as plsc

CHUNK_LEN = 8192
HIDDEN = 128
N_VARS_STATIC = 10000

NC = 2
NS = 16
NW = NC * NS

SB = 128
N_SB = 80
PER_W = N_SB * SB
NP_PAD = NW * PER_W
NV_PAD = 10240
TILE_CHUNKS = NV_PAD // (NS * SB)


def _sc_partials(table, cidx, lpos, seg):
    mesh = plsc.VectorSubcoreMesh(core_axis_name="c", subcore_axis_name="s")

    @functools.partial(
        pl.kernel,
        mesh=mesh,
        out_type=(
            jax.ShapeDtypeStruct((NC, NV_PAD, HIDDEN), jnp.float32),
            jax.ShapeDtypeStruct((NC, NV_PAD, HIDDEN), jnp.float32),
        ),
        scratch_types=[
            pltpu.VMEM((SB,), jnp.int32),
            pltpu.VMEM((SB,), jnp.int32),
            pltpu.VMEM((SB,), jnp.int32),
            pltpu.VMEM((SB, HIDDEN), jnp.float32),
            pltpu.VMEM((SB,), jnp.int32),
            pltpu.VMEM((SB,), jnp.int32),
            pltpu.VMEM((SB,), jnp.int32),
            pltpu.VMEM((SB, HIDDEN), jnp.float32),
            pltpu.VMEM_SHARED((NV_PAD, HIDDEN), jnp.float32),
            pltpu.SemaphoreType.DMA,
        ],
    )
    def k(table_h, cidx_h, lpos_h, seg_h, out_s_h, out_c_h,
          fidx_v, lpos_v, seg_v, rows_v,
          fidx2_v, lpos2_v, seg2_v, rows2_v, acc_s, sem):
        c = lax.axis_index("c")
        s = lax.axis_index("s")
        wid = c * NS + s
        base = wid * N_SB
        iota16 = lax.iota(jnp.int32, 16)

        def fill_rows(val):
            vec = jnp.full((16,), val, jnp.float32)

            def fill(r, carry):
                for v in range(HIDDEN // 16):
                    rows_v[r, pl.ds(v * 16, 16)] = vec
                return carry

            lax.fori_loop(0, SB, fill, 0)

        def make_idx(t):
            base_r = s * (TILE_CHUNKS * SB) + t * SB
            for v in range(SB // 16):
                fidx_v[pl.ds(v * 16, 16)] = iota16 + (base_r + v * 16)

        def zero_acc():
            fill_rows(0.0)
            for t in range(TILE_CHUNKS):
                make_idx(t)
                pltpu.sync_copy(rows_v, acc_s.at[fidx_v])

        def writeout(dst_h):
            for t in range(TILE_CHUNKS):
                make_idx(t)
                osl = pl.ds(s * (TILE_CHUNKS * SB) + t * SB, SB)
                pltpu.sync_copy(acc_s.at[fidx_v], rows_v)
                pltpu.sync_copy(rows_v, dst_h.at[c, osl])

        zero_acc()
        plsc.subcore_barrier()

        bufs = ((fidx_v, lpos_v, seg_v, rows_v),
                (fidx2_v, lpos2_v, seg2_v, rows2_v))

        def load_and_fire(row, f_v, l_v, s_v, r_v):
            pltpu.sync_copy(cidx_h.at[row], f_v)
            pltpu.sync_copy(lpos_h.at[row], l_v)
            pltpu.sync_copy(seg_h.at[row], s_v)
            for v in range(SB // 16):
                vsl = pl.ds(v * 16, 16)
                f_v[vsl] = f_v[vsl] * CHUNK_LEN + l_v[vsl]
            pltpu.async_copy(table_h.at[f_v], r_v, sem)

        def drain(r_v):
            pltpu.make_async_copy(table_h.at[pl.ds(0, SB)], r_v, sem).wait()

        for b in range(2):
            load_and_fire(base + b, *bufs[b])

        @pl.loop(0, N_SB // 2)
        def body(gi):
            blk = 2 * gi
            for b in range(2):
                f_v, l_v, s_v, r_v = bufs[b]
                drain(r_v)
                pltpu.sync_copy(r_v, acc_s.at[s_v], add=True)
                load_and_fire(base + blk + b + 2, f_v, l_v, s_v, r_v)

        for b in range(2):
            drain(bufs[b][3])

        plsc.subcore_barrier()
        writeout(out_s_h)
        plsc.subcore_barrier()

        zero_acc()
        plsc.subcore_barrier()
        fill_rows(1.0)

        @pl.loop(0, N_SB)
        def body2(sb):
            pltpu.sync_copy(seg_h.at[base + sb], seg_v)
            pltpu.sync_copy(rows_v, acc_s.at[seg_v], add=True)

        plsc.subcore_barrier()
        writeout(out_c_h)

    return k(table, cidx, lpos, seg)


def _combine(sums, cnts):
    g = 16
    r = NV_PAD // g

    def body(s_ref, c_ref, o_ref):
        ssum = s_ref[0] + s_ref[1]
        csum = c_ref[0] + c_ref[1]
        o_ref[...] = ssum / jnp.maximum(csum, 1.0)

    return pl.pallas_call(
        body,
        grid=(g,),
        in_specs=[
            pl.BlockSpec((NC, r, HIDDEN), lambda i: (0, i, 0)),
            pl.BlockSpec((NC, r, HIDDEN), lambda i: (0, i, 0)),
        ],
        out_specs=pl.BlockSpec((r, HIDDEN), lambda i: (i, 0)),
        out_shape=jax.ShapeDtypeStruct((NV_PAD, HIDDEN), jnp.float32),
    )(sums, cnts)


def kernel(chunk_hiddens, chunk_idx, local_pos, segment_ids, n_vars):
    n_chunks, chunk_len, hidden = chunk_hiddens.shape
    table = chunk_hiddens.reshape(n_chunks * chunk_len, hidden)

    n_pos = chunk_idx.shape[0]
    pad = NP_PAD + 2 * SB - n_pos
    cidx = jnp.pad(chunk_idx.astype(jnp.int32), (0, pad)).reshape(-1, SB)
    lpos = jnp.pad(local_pos.astype(jnp.int32), (0, pad)).reshape(-1, SB)
    seg = jnp.pad(
        segment_ids.astype(jnp.int32), (0, pad),
        constant_values=N_VARS_STATIC,
    ).reshape(-1, SB)

    sums, cnts = _sc_partials(table, cidx, lpos, seg)
    out = _combine(sums, cnts)
    return out[:N_VARS_STATIC]
